# Initial kernel scaffold; baseline (speedup 1.0000x reference)
import jax
import jax.numpy as jnp
from jax.experimental import pallas as pl


def kernel(noise, xt, timestep, alphas_cumprod):
    raise NotImplementedError("write your pallas kernel here")


# speedup vs baseline: 1.3178x; 1.3178x over previous
import functools

import jax
import jax.numpy as jnp
from jax import lax
from jax.experimental import pallas as pl
from jax.experimental.pallas import tpu as pltpu
from jax.experimental.pallas import tpu_sc as plsc

B, C, H, W = 16, 16, 128, 128
N = C * H * W
TOTAL = B * N
NC, NS, LANES = 2, 16, 16
NW = NC * NS
PER_WORKER = TOTAL // NW
CHUNK = 16384
NCHUNK = PER_WORKER // CHUNK
AC_PAD = 1024


def _babylonian_sqrt(x):
    y = jnp.full(x.shape, jnp.float32(1.0))
    for _ in range(15):
        y = jnp.float32(0.5) * (y + x / y)
    return y


def _sc_body(noise_hbm, xt_hbm, ts_hbm, ac_hbm, out_hbm,
             ts_v, ac_v, nb0, xb0, ob0, nb1, xb1, ob1,
             si0, si1, so0, so1):
    wid = lax.axis_index("s") * NC + lax.axis_index("c")
    base = wid * PER_WORKER

    pltpu.sync_copy(ts_hbm, ts_v)
    pltpu.sync_copy(ac_hbm, ac_v)
    t_b = ts_v[pl.ds(wid // 2, LANES)][0]
    a_scalar = ac_v[pl.ds(t_b, LANES)][0]
    a = jnp.full((LANES,), a_scalar, dtype=jnp.float32)
    sqrt_a = _babylonian_sqrt(a)
    sqrt_beta = _babylonian_sqrt(jnp.float32(1.0) - a)
    s1 = jnp.float32(1.0) / sqrt_a
    s2 = sqrt_beta / sqrt_a

    bufs = ((nb0, xb0, ob0, si0, so0), (nb1, xb1, ob1, si1, so1))
    in_handles = {}
    out_handles = {}

    def start_in(k):
        nb, xb, _, si, _ = bufs[k % 2]
        off = base + k * CHUNK
        h1 = pltpu.async_copy(noise_hbm.at[pl.ds(off, CHUNK)], nb, si)
        h2 = pltpu.async_copy(xt_hbm.at[pl.ds(off, CHUNK)], xb, si)
        in_handles[k] = (h1, h2)

    def compute(k):
        nb, xb, ob, _, _ = bufs[k % 2]

        @plsc.parallel_loop(jnp.int32(0), jnp.int32(CHUNK), jnp.int32(LANES),
                            unroll=8)
        def _(i):
            sl = pl.ds(i, LANES)
            ob[sl] = xb[sl] * s1 - nb[sl] * s2

    def start_out(k):
        _, _, ob, _, so = bufs[k % 2]
        off = base + k * CHUNK
        out_handles[k] = pltpu.async_copy(ob, out_hbm.at[pl.ds(off, CHUNK)], so)

    start_in(0)
    for k in range(NCHUNK):
        if k + 1 < NCHUNK:
            start_in(k + 1)
        for h in in_handles.pop(k):
            h.wait()
        if k >= 2:
            out_handles.pop(k - 2).wait()
        compute(k)
        start_out(k)
    for k in sorted(out_handles):
        out_handles[k].wait()


_sc_call = functools.partial(
    pl.kernel,
    out_type=jax.ShapeDtypeStruct((TOTAL,), jnp.float32),
    mesh=plsc.VectorSubcoreMesh(
        core_axis_name="c", subcore_axis_name="s",
        num_cores=NC, num_subcores=NS),
    scratch_types=[
        pltpu.VMEM((2 * B,), jnp.int32),
        pltpu.VMEM((AC_PAD,), jnp.float32),
        pltpu.VMEM((CHUNK,), jnp.float32),
        pltpu.VMEM((CHUNK,), jnp.float32),
        pltpu.VMEM((CHUNK,), jnp.float32),
        pltpu.VMEM((CHUNK,), jnp.float32),
        pltpu.VMEM((CHUNK,), jnp.float32),
        pltpu.VMEM((CHUNK,), jnp.float32),
        pltpu.SemaphoreType.DMA,
        pltpu.SemaphoreType.DMA,
        pltpu.SemaphoreType.DMA,
        pltpu.SemaphoreType.DMA,
    ],
)(_sc_body)


def kernel(noise, xt, timestep, alphas_cumprod):
    noise_flat = noise.reshape(TOTAL)
    xt_flat = xt.reshape(TOTAL)
    ts32 = jnp.concatenate(
        [timestep.astype(jnp.int32), jnp.zeros((B,), dtype=jnp.int32)])
    ac = alphas_cumprod.astype(jnp.float32)
    ac_padded = jnp.concatenate(
        [ac, jnp.ones((AC_PAD - ac.shape[0],), dtype=jnp.float32)])
    out = _sc_call(noise_flat, xt_flat, ts32, ac_padded)
    return out.reshape(B, C, H, W)
